# Initial kernel scaffold; baseline (speedup 1.0000x reference)
import jax
import jax.numpy as jnp
from jax.experimental import pallas as pl


def kernel(x_game, x_state, W_g, Wih_g, Whh_g, bih_g, bhh_g, W_s, Wih_s, Whh_s, bih_s, bhh_s, Wl1, bl1, Wr1, Wl2, bl2, Wr2, W1, b1, W2, b2, edge_index_gg, edge_index_ss, edge_index_hist, edge_index_in):
    raise NotImplementedError("write your pallas kernel here")


# speedup vs baseline: 1.7830x; 1.7830x over previous
import functools
import jax
import jax.numpy as jnp
from jax import lax
from jax.experimental import pallas as pl
from jax.experimental.pallas import tpu as pltpu
from jax.experimental.pallas import tpu_sc as plsc

H = 256
HH = 128
N = 10000
E = 160000
R = 10240
EP = 163840
CHUNK = 128
N_CHUNK = EP // (16 * CHUNK)
ROWS_PER_TILE = R // 16


def _sc_body_counts(with_counts, tab1, tab2, srcs, dsts, z128, ones128,
                    *refs):
    if with_counts:
        (acc1, acc2, cntO, srcv, dstv, rowsv, onesv, accS, gsem) = refs
    else:
        (acc1, acc2, srcv, dstv, rowsv, onesv, accS, gsem) = refs
    c = lax.axis_index("c")
    s = lax.axis_index("s")
    r0 = s * ROWS_PER_TILE
    e0 = s * (EP // 16)

    def run_phase(gather_tab, src_ref, dst_ref, out_ref):
        pltpu.sync_copy(z128.at[pl.ds(r0, ROWS_PER_TILE)],
                        accS.at[pl.ds(r0, ROWS_PER_TILE)])
        plsc.subcore_barrier()

        def chunk_body(i, carry):
            base = e0 + i * CHUNK
            pltpu.sync_copy(dst_ref.at[pl.ds(base, CHUNK)], dstv)
            if gather_tab is not None:
                pltpu.sync_copy(src_ref.at[pl.ds(base, CHUNK)], srcv)
                pltpu.async_copy(gather_tab.at[srcv], rowsv, gsem).wait()
                pltpu.sync_copy(rowsv, accS.at[dstv], add=True)
            else:
                pltpu.sync_copy(onesv, accS.at[dstv], add=True)
            return carry

        lax.fori_loop(0, N_CHUNK, chunk_body, 0)
        plsc.subcore_barrier()
        pltpu.sync_copy(accS.at[pl.ds(r0, ROWS_PER_TILE)],
                        out_ref.at[pl.ds(r0, ROWS_PER_TILE)])
        plsc.subcore_barrier()

    run_phase(tab1.at[c], srcs.at[0], dsts.at[0], acc1.at[c])
    run_phase(tab2.at[c], srcs.at[1], dsts.at[1], acc2.at[c])
    if with_counts:
        pltpu.sync_copy(ones128, onesv)
        run_phase(None, None, dsts.at[c], cntO.at[c])


@functools.lru_cache(maxsize=2)
def _make_sc_segsum2(with_counts):
  out_type = [
      jax.ShapeDtypeStruct((2, R, HH), jnp.float32),
      jax.ShapeDtypeStruct((2, R, HH), jnp.float32),
  ]
  if with_counts:
      out_type.append(jax.ShapeDtypeStruct((2, R, HH), jnp.float32))
  return functools.partial(
    pl.kernel,
    out_type=out_type,
    mesh=plsc.VectorSubcoreMesh(core_axis_name="c", subcore_axis_name="s",
                                num_cores=2, num_subcores=16),
    scratch_types=[
        pltpu.VMEM((CHUNK,), jnp.int32),
        pltpu.VMEM((CHUNK,), jnp.int32),
        pltpu.VMEM((CHUNK, HH), jnp.float32),
        pltpu.VMEM((CHUNK, HH), jnp.float32),
        pltpu.VMEM_SHARED((R, HH), jnp.float32),
        pltpu.SemaphoreType.DMA,
    ],
  )(functools.partial(_sc_body_counts, with_counts))


def _pad_edges(ei):
    src = jnp.concatenate([ei[0], jnp.zeros((EP - E,), jnp.int32)])
    dst = jnp.concatenate([ei[1], jnp.full((EP - E,), N, jnp.int32)])
    return src, dst


def _segment_sums(tab1, tab2, ei1, ei2, consts, with_counts=False):
    src1, dst1 = _pad_edges(ei1)
    src2, dst2 = _pad_edges(ei2)
    srcs = jnp.stack([src1, src2])
    dsts = jnp.stack([dst1, dst2])
    z128, ones128 = consts
    return _make_sc_segsum2(with_counts)(tab1, tab2, srcs, dsts,
                                         z128, ones128)


def _mm_body(xg, xs, Wg, Ws, mg, ms):
    g = jnp.dot(xg[:], Wg[:], preferred_element_type=jnp.float32)
    mg[0] = g[:, :HH]
    mg[1] = g[:, HH:]
    s = jnp.dot(xs[:], Ws[:], preferred_element_type=jnp.float32)
    ms[0] = s[:, :HH]
    ms[1] = s[:, HH:]


def _tc_mm(xg, xs, Wg, Ws):
    B = 1000
    grid = (N // B,)
    bs_in = pl.BlockSpec((B, H), lambda i: (i, 0))
    bs_w = pl.BlockSpec((H, H), lambda i: (0, 0))
    bs_out = pl.BlockSpec((2, B, HH), lambda i: (0, i, 0))
    out_shape = [jax.ShapeDtypeStruct((2, N, HH), jnp.float32)] * 2
    return pl.pallas_call(
        _mm_body, grid=grid,
        in_specs=[bs_in, bs_in, bs_w, bs_w],
        out_specs=[bs_out] * 2,
        out_shape=out_shape,
    )(xg, xs, Wg, Ws)


def _gru_body(agl, agh, xg, asl, ash, xs,
              WihTg_lo, WihTg_hi, WhhTg, bihg, bhhg,
              WihTs_lo, WihTs_hi, WhhTs, bihs, bhhs,
              game_out, state_out):
    def gru(a_lo, a_hi, x, WT_lo, WT_hi, WhhT, bih, bhh):
        gi = (jnp.dot(a_lo[:], WT_lo[:], preferred_element_type=jnp.float32)
              + jnp.dot(a_hi[:], WT_hi[:], preferred_element_type=jnp.float32)
              + bih[:])
        gh = jnp.dot(x[:], WhhT[:], preferred_element_type=jnp.float32) + bhh[:]
        r = jax.nn.sigmoid(gi[:, :H] + gh[:, :H])
        z = jax.nn.sigmoid(gi[:, H:2 * H] + gh[:, H:2 * H])
        n = jnp.tanh(gi[:, 2 * H:] + r * gh[:, 2 * H:])
        return jax.nn.relu((1.0 - z) * n + z * x[:])

    g = gru(agl, agh, xg, WihTg_lo, WihTg_hi, WhhTg, bihg, bhhg)
    game_out[0] = g[:, :HH]
    game_out[1] = g[:, HH:]
    state_out[:] = gru(asl, ash, xs, WihTs_lo, WihTs_hi, WhhTs, bihs, bhhs)


def _tc_gru(agl, agh, xg, asl, ash, xs, wg, ws):
    B = 1000
    grid = (N // B,)
    bs_h = pl.BlockSpec((B, HH), lambda i: (i, 0))
    bs_f = pl.BlockSpec((B, H), lambda i: (i, 0))
    bs_wih = pl.BlockSpec((HH, 3 * H), lambda i: (0, 0))
    bs_whh = pl.BlockSpec((H, 3 * H), lambda i: (0, 0))
    bs_b = pl.BlockSpec((1, 3 * H), lambda i: (0, 0))
    bs_tab = pl.BlockSpec((2, B, HH), lambda i: (0, i, 0))
    out_shape = [jax.ShapeDtypeStruct((2, N, HH), jnp.float32),
                 jax.ShapeDtypeStruct((N, H), jnp.float32)]
    return pl.pallas_call(
        _gru_body, grid=grid,
        in_specs=[bs_h, bs_h, bs_f, bs_h, bs_h, bs_f,
                  bs_wih, bs_wih, bs_whh, bs_b, bs_b,
                  bs_wih, bs_wih, bs_whh, bs_b, bs_b],
        out_specs=[bs_tab, bs_f],
        out_shape=out_shape,
    )(agl, agh, xg, asl, ash, xs, *wg, *ws)


def _sage_body(s1l, s1h, c1, s2l, s2h, c2, state,
               Wl1T_lo, Wl1T_hi, Wr1T, bl1,
               Wl2T_lo, Wl2T_hi, Wr2T, bl2,
               W1T, b1, W2T, b2, out):
    cnt1 = jnp.maximum(c1[:, 0:1], 1.0)
    m1l = s1l[:] / cnt1
    m1h = s1h[:] / cnt1
    common1 = jax.nn.relu(
        jnp.dot(m1l, Wl1T_lo[:], preferred_element_type=jnp.float32)
        + jnp.dot(m1h, Wl1T_hi[:], preferred_element_type=jnp.float32)
        + bl1[:]
        + jnp.dot(state[:], Wr1T[:], preferred_element_type=jnp.float32))
    cnt2 = jnp.maximum(c2[:, 0:1], 1.0)
    m2l = s2l[:] / cnt2
    m2h = s2h[:] / cnt2
    common2 = jax.nn.relu(
        jnp.dot(m2l, Wl2T_lo[:], preferred_element_type=jnp.float32)
        + jnp.dot(m2h, Wl2T_hi[:], preferred_element_type=jnp.float32)
        + bl2[:]
        + jnp.dot(common1, Wr2T[:], preferred_element_type=jnp.float32))
    h = jax.nn.relu(jnp.dot(common2, W1T[:], preferred_element_type=jnp.float32)
                    + b1[:])
    out[:] = jnp.dot(h, W2T[:], preferred_element_type=jnp.float32) + b2[:]


def _tc_sage(s1l, s1h, c1, s2l, s2h, c2, state, weights):
    B = 1000
    grid = (N // B,)
    bs_h = pl.BlockSpec((B, HH), lambda i: (i, 0))
    bs_c = pl.BlockSpec((B, HH), lambda i: (i, 0))
    bs_f = pl.BlockSpec((B, H), lambda i: (i, 0))
    bs_whalf = pl.BlockSpec((HH, H), lambda i: (0, 0))
    bs_wfull = pl.BlockSpec((H, H), lambda i: (0, 0))
    bs_b = pl.BlockSpec((1, H), lambda i: (0, 0))
    bs_w1 = pl.BlockSpec((H, 20), lambda i: (0, 0))
    bs_b1 = pl.BlockSpec((1, 20), lambda i: (0, 0))
    bs_w2 = pl.BlockSpec((20, 1), lambda i: (0, 0))
    bs_b2 = pl.BlockSpec((1, 1), lambda i: (0, 0))
    bs_out = pl.BlockSpec((B, 1), lambda i: (i, 0))
    return pl.pallas_call(
        _sage_body, grid=grid,
        in_specs=[bs_h, bs_h, bs_c, bs_h, bs_h, bs_c, bs_f,
                  bs_whalf, bs_whalf, bs_wfull, bs_b,
                  bs_whalf, bs_whalf, bs_wfull, bs_b,
                  bs_w1, bs_b1, bs_w2, bs_b2],
        out_specs=bs_out,
        out_shape=jax.ShapeDtypeStruct((N, 1), jnp.float32),
    )(s1l, s1h, c1, s2l, s2h, c2, state, *weights)


def kernel(x_game, x_state, W_g, Wih_g, Whh_g, bih_g, bhh_g,
           W_s, Wih_s, Whh_s, bih_s, bhh_s,
           Wl1, bl1, Wr1, Wl2, bl2, Wr2,
           W1, b1, W2, b2,
           edge_index_gg, edge_index_ss, edge_index_hist, edge_index_in):
    f32 = jnp.float32
    consts = (jnp.zeros((R, HH), f32), jnp.ones((CHUNK, HH), f32))

    mg, ms = _tc_mm(x_game, x_state, W_g, W_s)

    agg_g, agg_s = _segment_sums(
        mg, ms, edge_index_gg, edge_index_ss, consts)

    wg = (Wih_g.T[:HH], Wih_g.T[HH:], Whh_g.T,
          bih_g.reshape(1, -1), bhh_g.reshape(1, -1))
    ws = (Wih_s.T[:HH], Wih_s.T[HH:], Whh_s.T,
          bih_s.reshape(1, -1), bhh_s.reshape(1, -1))
    game_tab, state_x = _tc_gru(
        agg_g[0, :N], agg_g[1, :N], x_game,
        agg_s[0, :N], agg_s[1, :N], x_state, wg, ws)

    sum1, sum2, cntO = _segment_sums(
        game_tab, game_tab, edge_index_hist, edge_index_in, consts,
        with_counts=True)
    s1l, s1h = sum1[0], sum1[1]
    s2l, s2h = sum2[0], sum2[1]
    c1, c2 = cntO[0], cntO[1]

    weights = (Wl1.T[:HH], Wl1.T[HH:], Wr1.T, bl1.reshape(1, -1),
               Wl2.T[:HH], Wl2.T[HH:], Wr2.T, bl2.reshape(1, -1),
               W1.T, b1.reshape(1, -1), W2.T, b2.reshape(1, -1))
    return _tc_sage(s1l[:N], s1h[:N], c1[:N], s2l[:N], s2h[:N], c2[:N],
                    state_x, weights)


# speedup vs baseline: 2.2460x; 1.2597x over previous
import functools
import jax
import jax.numpy as jnp
from jax import lax
from jax.experimental import pallas as pl
from jax.experimental.pallas import tpu as pltpu
from jax.experimental.pallas import tpu_sc as plsc

H = 256
HH = 128
N = 10000
E = 160000
R = 10240
EP = 163840
CHUNK = 128
N_CHUNK = EP // (16 * CHUNK)
HALF_CHUNKS = N_CHUNK // 2
ROWS_PER_TILE = R // 16


def _sc_body_counts(with_counts, tab1, tab2, srcs, dsts, z128, ones128,
                    *refs):
    if with_counts:
        (acc1, acc2, cntO, srcA, dstA, rows0, rows1, accS,
         gsem0, gsem1, ssem0, ssem1) = refs
    else:
        (acc1, acc2, srcA, dstA, rows0, rows1, accS,
         gsem0, gsem1, ssem0, ssem1) = refs
    c = lax.axis_index("c")
    s = lax.axis_index("s")
    r0 = s * ROWS_PER_TILE
    ch0 = s * N_CHUNK
    rows = (rows0, rows1)
    gsems = (gsem0, gsem1)
    ssems = (ssem0, ssem1)

    def run_phase(gather_tab, src3, dst3, out_ref):
        pltpu.sync_copy(z128.at[pl.ds(r0, ROWS_PER_TILE)],
                        accS.at[pl.ds(r0, ROWS_PER_TILE)])
        plsc.subcore_barrier()
        for h in range(2):
            run_half(gather_tab, src3, dst3, ch0 + h * HALF_CHUNKS)
        plsc.subcore_barrier()
        pltpu.sync_copy(accS.at[pl.ds(r0, ROWS_PER_TILE)],
                        out_ref.at[pl.ds(r0, ROWS_PER_TILE)])
        plsc.subcore_barrier()

    def run_half(gather_tab, src3, dst3, chbase):
        if gather_tab is not None:
            pltpu.sync_copy(src3.at[pl.ds(chbase, HALF_CHUNKS)], srcA)
        pltpu.sync_copy(dst3.at[pl.ds(chbase, HALF_CHUNKS)], dstA)

        NJ = HALF_CHUNKS // 2

        if gather_tab is not None:
            def g_desc(i, b):
                return pltpu.make_async_copy(gather_tab.at[srcA.at[i]],
                                             rows[b], gsems[b])

            def s_desc(i, b):
                return pltpu.make_async_copy(rows[b], accS.at[dstA.at[i]],
                                             ssems[b])

            g_desc(0, 0).start()

            def body(j, carry):
                i0 = j * 2
                g_desc(i0, 0).wait()

                @pl.when(j > 0)
                def _():
                    s_desc(i0 - 1, 1).wait()

                g_desc(i0 + 1, 1).start()
                s_desc(i0, 0).start(add=True)
                g_desc(i0 + 1, 1).wait()
                s_desc(i0, 0).wait()

                @pl.when(j < NJ - 1)
                def _():
                    g_desc(i0 + 2, 0).start()

                s_desc(i0 + 1, 1).start(add=True)
                return carry

            lax.fori_loop(0, NJ, body, 0)
            pltpu.make_async_copy(rows[1], accS.at[dstA.at[HALF_CHUNKS - 1]],
                                  ssems[1]).wait()
        else:
            def s_desc(i, b):
                return pltpu.make_async_copy(rows0, accS.at[dstA.at[i]],
                                             ssems[b])

            def body(j, carry):
                i0 = j * 2
                s_desc(i0, 0).start(add=True)

                @pl.when(j > 0)
                def _():
                    s_desc(i0 - 1, 1).wait()

                s_desc(i0 + 1, 1).start(add=True)
                s_desc(i0, 0).wait()
                return carry

            lax.fori_loop(0, NJ, body, 0)
            s_desc(HALF_CHUNKS - 1, 1).wait()

    run_phase(tab1.at[c], srcs.at[0], dsts.at[0], acc1.at[c])
    run_phase(tab2.at[c], srcs.at[1], dsts.at[1], acc2.at[c])
    if with_counts:
        pltpu.sync_copy(ones128, rows0)
        run_phase(None, None, dsts.at[c], cntO.at[c])


@functools.lru_cache(maxsize=2)
def _make_sc_segsum2(with_counts):
  out_type = [
      jax.ShapeDtypeStruct((2, R, HH), jnp.float32),
      jax.ShapeDtypeStruct((2, R, HH), jnp.float32),
  ]
  if with_counts:
      out_type.append(jax.ShapeDtypeStruct((2, R, HH), jnp.float32))
  return functools.partial(
    pl.kernel,
    out_type=out_type,
    mesh=plsc.VectorSubcoreMesh(core_axis_name="c", subcore_axis_name="s",
                                num_cores=2, num_subcores=16),
    scratch_types=[
        pltpu.VMEM((HALF_CHUNKS, CHUNK), jnp.int32),
        pltpu.VMEM((HALF_CHUNKS, CHUNK), jnp.int32),
        pltpu.VMEM((CHUNK, HH), jnp.float32),
        pltpu.VMEM((CHUNK, HH), jnp.float32),
        pltpu.VMEM_SHARED((R, HH), jnp.float32),
        pltpu.SemaphoreType.DMA,
        pltpu.SemaphoreType.DMA,
        pltpu.SemaphoreType.DMA,
        pltpu.SemaphoreType.DMA,
    ],
  )(functools.partial(_sc_body_counts, with_counts))


def _pad_edges(ei):
    src = jnp.concatenate([ei[0], jnp.zeros((EP - E,), jnp.int32)])
    dst = jnp.concatenate([ei[1], jnp.full((EP - E,), N, jnp.int32)])
    return src, dst


def _segment_sums(tab1, tab2, ei1, ei2, consts, with_counts=False):
    src1, dst1 = _pad_edges(ei1)
    src2, dst2 = _pad_edges(ei2)
    srcs = jnp.stack([src1, src2]).reshape(2, EP // CHUNK, CHUNK)
    dsts = jnp.stack([dst1, dst2]).reshape(2, EP // CHUNK, CHUNK)
    z128, ones128 = consts
    return _make_sc_segsum2(with_counts)(tab1, tab2, srcs, dsts,
                                         z128, ones128)


def _mm_body(xg, xs, Wg, Ws, mg, ms):
    g = jnp.dot(xg[:], Wg[:], preferred_element_type=jnp.float32)
    mg[0] = g[:, :HH]
    mg[1] = g[:, HH:]
    s = jnp.dot(xs[:], Ws[:], preferred_element_type=jnp.float32)
    ms[0] = s[:, :HH]
    ms[1] = s[:, HH:]


def _tc_mm(xg, xs, Wg, Ws):
    B = 1000
    grid = (N // B,)
    bs_in = pl.BlockSpec((B, H), lambda i: (i, 0))
    bs_w = pl.BlockSpec((H, H), lambda i: (0, 0))
    bs_out = pl.BlockSpec((2, B, HH), lambda i: (0, i, 0))
    out_shape = [jax.ShapeDtypeStruct((2, N, HH), jnp.float32)] * 2
    return pl.pallas_call(
        _mm_body, grid=grid,
        in_specs=[bs_in, bs_in, bs_w, bs_w],
        out_specs=[bs_out] * 2,
        out_shape=out_shape,
    )(xg, xs, Wg, Ws)


def _gru_body(agl, agh, xg, asl, ash, xs,
              WihTg_lo, WihTg_hi, WhhTg, bihg, bhhg,
              WihTs_lo, WihTs_hi, WhhTs, bihs, bhhs,
              game_out, state_out):
    def gru(a_lo, a_hi, x, WT_lo, WT_hi, WhhT, bih, bhh):
        gi = (jnp.dot(a_lo[:], WT_lo[:], preferred_element_type=jnp.float32)
              + jnp.dot(a_hi[:], WT_hi[:], preferred_element_type=jnp.float32)
              + bih[:])
        gh = jnp.dot(x[:], WhhT[:], preferred_element_type=jnp.float32) + bhh[:]
        r = jax.nn.sigmoid(gi[:, :H] + gh[:, :H])
        z = jax.nn.sigmoid(gi[:, H:2 * H] + gh[:, H:2 * H])
        n = jnp.tanh(gi[:, 2 * H:] + r * gh[:, 2 * H:])
        return jax.nn.relu((1.0 - z) * n + z * x[:])

    g = gru(agl, agh, xg, WihTg_lo, WihTg_hi, WhhTg, bihg, bhhg)
    game_out[0] = g[:, :HH]
    game_out[1] = g[:, HH:]
    state_out[:] = gru(asl, ash, xs, WihTs_lo, WihTs_hi, WhhTs, bihs, bhhs)


def _tc_gru(agl, agh, xg, asl, ash, xs, wg, ws):
    B = 1000
    grid = (N // B,)
    bs_h = pl.BlockSpec((B, HH), lambda i: (i, 0))
    bs_f = pl.BlockSpec((B, H), lambda i: (i, 0))
    bs_wih = pl.BlockSpec((HH, 3 * H), lambda i: (0, 0))
    bs_whh = pl.BlockSpec((H, 3 * H), lambda i: (0, 0))
    bs_b = pl.BlockSpec((1, 3 * H), lambda i: (0, 0))
    bs_tab = pl.BlockSpec((2, B, HH), lambda i: (0, i, 0))
    out_shape = [jax.ShapeDtypeStruct((2, N, HH), jnp.float32),
                 jax.ShapeDtypeStruct((N, H), jnp.float32)]
    return pl.pallas_call(
        _gru_body, grid=grid,
        in_specs=[bs_h, bs_h, bs_f, bs_h, bs_h, bs_f,
                  bs_wih, bs_wih, bs_whh, bs_b, bs_b,
                  bs_wih, bs_wih, bs_whh, bs_b, bs_b],
        out_specs=[bs_tab, bs_f],
        out_shape=out_shape,
    )(agl, agh, xg, asl, ash, xs, *wg, *ws)


def _sage_body(s1l, s1h, c1, s2l, s2h, c2, state,
               Wl1T_lo, Wl1T_hi, Wr1T, bl1,
               Wl2T_lo, Wl2T_hi, Wr2T, bl2,
               W1T, b1, W2T, b2, out):
    cnt1 = jnp.maximum(c1[:, 0:1], 1.0)
    m1l = s1l[:] / cnt1
    m1h = s1h[:] / cnt1
    common1 = jax.nn.relu(
        jnp.dot(m1l, Wl1T_lo[:], preferred_element_type=jnp.float32)
        + jnp.dot(m1h, Wl1T_hi[:], preferred_element_type=jnp.float32)
        + bl1[:]
        + jnp.dot(state[:], Wr1T[:], preferred_element_type=jnp.float32))
    cnt2 = jnp.maximum(c2[:, 0:1], 1.0)
    m2l = s2l[:] / cnt2
    m2h = s2h[:] / cnt2
    common2 = jax.nn.relu(
        jnp.dot(m2l, Wl2T_lo[:], preferred_element_type=jnp.float32)
        + jnp.dot(m2h, Wl2T_hi[:], preferred_element_type=jnp.float32)
        + bl2[:]
        + jnp.dot(common1, Wr2T[:], preferred_element_type=jnp.float32))
    h = jax.nn.relu(jnp.dot(common2, W1T[:], preferred_element_type=jnp.float32)
                    + b1[:])
    out[:] = jnp.dot(h, W2T[:], preferred_element_type=jnp.float32) + b2[:]


def _tc_sage(s1l, s1h, c1, s2l, s2h, c2, state, weights):
    B = 1000
    grid = (N // B,)
    bs_h = pl.BlockSpec((B, HH), lambda i: (i, 0))
    bs_c = pl.BlockSpec((B, HH), lambda i: (i, 0))
    bs_f = pl.BlockSpec((B, H), lambda i: (i, 0))
    bs_whalf = pl.BlockSpec((HH, H), lambda i: (0, 0))
    bs_wfull = pl.BlockSpec((H, H), lambda i: (0, 0))
    bs_b = pl.BlockSpec((1, H), lambda i: (0, 0))
    bs_w1 = pl.BlockSpec((H, 20), lambda i: (0, 0))
    bs_b1 = pl.BlockSpec((1, 20), lambda i: (0, 0))
    bs_w2 = pl.BlockSpec((20, 1), lambda i: (0, 0))
    bs_b2 = pl.BlockSpec((1, 1), lambda i: (0, 0))
    bs_out = pl.BlockSpec((B, 1), lambda i: (i, 0))
    return pl.pallas_call(
        _sage_body, grid=grid,
        in_specs=[bs_h, bs_h, bs_c, bs_h, bs_h, bs_c, bs_f,
                  bs_whalf, bs_whalf, bs_wfull, bs_b,
                  bs_whalf, bs_whalf, bs_wfull, bs_b,
                  bs_w1, bs_b1, bs_w2, bs_b2],
        out_specs=bs_out,
        out_shape=jax.ShapeDtypeStruct((N, 1), jnp.float32),
    )(s1l, s1h, c1, s2l, s2h, c2, state, *weights)


def kernel(x_game, x_state, W_g, Wih_g, Whh_g, bih_g, bhh_g,
           W_s, Wih_s, Whh_s, bih_s, bhh_s,
           Wl1, bl1, Wr1, Wl2, bl2, Wr2,
           W1, b1, W2, b2,
           edge_index_gg, edge_index_ss, edge_index_hist, edge_index_in):
    f32 = jnp.float32
    consts = (jnp.zeros((R, HH), f32), jnp.ones((CHUNK, HH), f32))

    mg, ms = _tc_mm(x_game, x_state, W_g, W_s)

    agg_g, agg_s = _segment_sums(
        mg, ms, edge_index_gg, edge_index_ss, consts)

    wg = (Wih_g.T[:HH], Wih_g.T[HH:], Whh_g.T,
          bih_g.reshape(1, -1), bhh_g.reshape(1, -1))
    ws = (Wih_s.T[:HH], Wih_s.T[HH:], Whh_s.T,
          bih_s.reshape(1, -1), bhh_s.reshape(1, -1))
    game_tab, state_x = _tc_gru(
        agg_g[0, :N], agg_g[1, :N], x_game,
        agg_s[0, :N], agg_s[1, :N], x_state, wg, ws)

    sum1, sum2, cntO = _segment_sums(
        game_tab, game_tab, edge_index_hist, edge_index_in, consts,
        with_counts=True)
    s1l, s1h = sum1[0], sum1[1]
    s2l, s2h = sum2[0], sum2[1]
    c1, c2 = cntO[0], cntO[1]

    weights = (Wl1.T[:HH], Wl1.T[HH:], Wr1.T, bl1.reshape(1, -1),
               Wl2.T[:HH], Wl2.T[HH:], Wr2.T, bl2.reshape(1, -1),
               W1.T, b1.reshape(1, -1), W2.T, b2.reshape(1, -1))
    return _tc_sage(s1l[:N], s1h[:N], c1[:N], s2l[:N], s2h[:N], c2[:N],
                    state_x, weights)


# speedup vs baseline: 2.3502x; 1.0464x over previous
import functools
import jax
import jax.numpy as jnp
from jax import lax
from jax.experimental import pallas as pl
from jax.experimental.pallas import tpu as pltpu
from jax.experimental.pallas import tpu_sc as plsc

H = 256
HH = 128
N = 10000
E = 160000
R = 10240
EP = 163840
CHUNK = 128
N_CHUNK = EP // (16 * CHUNK)
HALF_CHUNKS = N_CHUNK // 2
ROWS_PER_TILE = R // 16


def _sc_body_counts(with_counts, tab1, tab2, srcs, dsts, z128, ones128,
                    *refs):
    if with_counts:
        (acc1, acc2, cntO, srcA, dstA, rows0, rows1, accS,
         gsem0, gsem1, ssem0, ssem1) = refs
    else:
        (acc1, acc2, srcA, dstA, rows0, rows1, accS,
         gsem0, gsem1, ssem0, ssem1) = refs
    c = lax.axis_index("c")
    s = lax.axis_index("s")
    r0 = s * ROWS_PER_TILE
    ch0 = s * N_CHUNK
    rows = (rows0, rows1)
    gsems = (gsem0, gsem1)
    ssems = (ssem0, ssem1)

    def run_phase(gather_tab, src3, dst3, out_ref):
        pltpu.sync_copy(z128.at[pl.ds(r0, ROWS_PER_TILE)],
                        accS.at[pl.ds(r0, ROWS_PER_TILE)])
        plsc.subcore_barrier()
        for h in range(2):
            run_half(gather_tab, src3, dst3, ch0 + h * HALF_CHUNKS)
        plsc.subcore_barrier()
        pltpu.sync_copy(accS.at[pl.ds(r0, ROWS_PER_TILE)],
                        out_ref.at[pl.ds(r0, ROWS_PER_TILE)])
        plsc.subcore_barrier()

    def run_half(gather_tab, src3, dst3, chbase):
        if gather_tab is not None:
            pltpu.sync_copy(src3.at[pl.ds(chbase, HALF_CHUNKS)], srcA)
        pltpu.sync_copy(dst3.at[pl.ds(chbase, HALF_CHUNKS)], dstA)

        NJ = HALF_CHUNKS // 2

        if gather_tab is not None:
            def g_desc(i, b):
                return pltpu.make_async_copy(gather_tab.at[srcA.at[i]],
                                             rows[b], gsems[b])

            def s_desc(i, b):
                return pltpu.make_async_copy(rows[b], accS.at[dstA.at[i]],
                                             ssems[b])

            g_desc(0, 0).start()
            g_desc(1, 1).start()

            def body(j, carry):
                i0 = j * 2
                g_desc(i0, 0).wait()
                s_desc(i0, 0).start(add=True)
                g_desc(i0 + 1, 1).wait()
                s_desc(i0, 0).wait()

                @pl.when(j < NJ - 1)
                def _():
                    g_desc(i0 + 2, 0).start()

                s_desc(i0 + 1, 1).start(add=True)
                s_desc(i0 + 1, 1).wait()

                @pl.when(j < NJ - 1)
                def _():
                    g_desc(i0 + 3, 1).start()

                return carry

            lax.fori_loop(0, NJ, body, 0)
        else:
            def s_desc(i, b):
                return pltpu.make_async_copy(rows0, accS.at[dstA.at[i]],
                                             ssems[b])

            def body(j, carry):
                i0 = j * 2

                @pl.when(j > 0)
                def _():
                    s_desc(i0 - 2, 0).wait()
                    s_desc(i0 - 1, 1).wait()

                s_desc(i0, 0).start(add=True)
                s_desc(i0 + 1, 1).start(add=True)
                return carry

            lax.fori_loop(0, NJ, body, 0)
            s_desc(HALF_CHUNKS - 2, 0).wait()
            s_desc(HALF_CHUNKS - 1, 1).wait()

    run_phase(tab1.at[c], srcs.at[0], dsts.at[0], acc1.at[c])
    run_phase(tab2.at[c], srcs.at[1], dsts.at[1], acc2.at[c])
    if with_counts:
        pltpu.sync_copy(ones128, rows0)
        run_phase(None, None, dsts.at[c], cntO.at[c])


@functools.lru_cache(maxsize=2)
def _make_sc_segsum2(with_counts):
  out_type = [
      jax.ShapeDtypeStruct((2, R, HH), jnp.float32),
      jax.ShapeDtypeStruct((2, R, HH), jnp.float32),
  ]
  if with_counts:
      out_type.append(jax.ShapeDtypeStruct((2, R, HH), jnp.float32))
  return functools.partial(
    pl.kernel,
    out_type=out_type,
    mesh=plsc.VectorSubcoreMesh(core_axis_name="c", subcore_axis_name="s",
                                num_cores=2, num_subcores=16),
    scratch_types=[
        pltpu.VMEM((HALF_CHUNKS, CHUNK), jnp.int32),
        pltpu.VMEM((HALF_CHUNKS, CHUNK), jnp.int32),
        pltpu.VMEM((CHUNK, HH), jnp.float32),
        pltpu.VMEM((CHUNK, HH), jnp.float32),
        pltpu.VMEM_SHARED((R, HH), jnp.float32),
        pltpu.SemaphoreType.DMA,
        pltpu.SemaphoreType.DMA,
        pltpu.SemaphoreType.DMA,
        pltpu.SemaphoreType.DMA,
    ],
  )(functools.partial(_sc_body_counts, with_counts))


def _pad_edges(ei):
    src = jnp.concatenate([ei[0], jnp.zeros((EP - E,), jnp.int32)])
    dst = jnp.concatenate([ei[1], jnp.full((EP - E,), N, jnp.int32)])
    return src, dst


def _segment_sums(tab1, tab2, ei1, ei2, consts, with_counts=False):
    src1, dst1 = _pad_edges(ei1)
    src2, dst2 = _pad_edges(ei2)
    srcs = jnp.stack([src1, src2]).reshape(2, EP // CHUNK, CHUNK)
    dsts = jnp.stack([dst1, dst2]).reshape(2, EP // CHUNK, CHUNK)
    z128, ones128 = consts
    return _make_sc_segsum2(with_counts)(tab1, tab2, srcs, dsts,
                                         z128, ones128)


def _mm_body(xg, xs, Wg, Ws, mg, ms):
    g = jnp.dot(xg[:], Wg[:], preferred_element_type=jnp.float32)
    mg[0] = g[:, :HH]
    mg[1] = g[:, HH:]
    s = jnp.dot(xs[:], Ws[:], preferred_element_type=jnp.float32)
    ms[0] = s[:, :HH]
    ms[1] = s[:, HH:]


def _tc_mm(xg, xs, Wg, Ws):
    B = 1000
    grid = (N // B,)
    bs_in = pl.BlockSpec((B, H), lambda i: (i, 0))
    bs_w = pl.BlockSpec((H, H), lambda i: (0, 0))
    bs_out = pl.BlockSpec((2, B, HH), lambda i: (0, i, 0))
    out_shape = [jax.ShapeDtypeStruct((2, N, HH), jnp.float32)] * 2
    return pl.pallas_call(
        _mm_body, grid=grid,
        in_specs=[bs_in, bs_in, bs_w, bs_w],
        out_specs=[bs_out] * 2,
        out_shape=out_shape,
    )(xg, xs, Wg, Ws)


def _gru_body(agl, agh, xg, asl, ash, xs,
              WihTg_lo, WihTg_hi, WhhTg, bihg, bhhg,
              WihTs_lo, WihTs_hi, WhhTs, bihs, bhhs,
              game_out, state_out):
    def gru(a_lo, a_hi, x, WT_lo, WT_hi, WhhT, bih, bhh):
        gi = (jnp.dot(a_lo[:], WT_lo[:], preferred_element_type=jnp.float32)
              + jnp.dot(a_hi[:], WT_hi[:], preferred_element_type=jnp.float32)
              + bih[:])
        gh = jnp.dot(x[:], WhhT[:], preferred_element_type=jnp.float32) + bhh[:]
        r = jax.nn.sigmoid(gi[:, :H] + gh[:, :H])
        z = jax.nn.sigmoid(gi[:, H:2 * H] + gh[:, H:2 * H])
        n = jnp.tanh(gi[:, 2 * H:] + r * gh[:, 2 * H:])
        return jax.nn.relu((1.0 - z) * n + z * x[:])

    g = gru(agl, agh, xg, WihTg_lo, WihTg_hi, WhhTg, bihg, bhhg)
    game_out[0] = g[:, :HH]
    game_out[1] = g[:, HH:]
    state_out[:] = gru(asl, ash, xs, WihTs_lo, WihTs_hi, WhhTs, bihs, bhhs)


def _tc_gru(agl, agh, xg, asl, ash, xs, wg, ws):
    B = 1000
    grid = (N // B,)
    bs_h = pl.BlockSpec((B, HH), lambda i: (i, 0))
    bs_f = pl.BlockSpec((B, H), lambda i: (i, 0))
    bs_wih = pl.BlockSpec((HH, 3 * H), lambda i: (0, 0))
    bs_whh = pl.BlockSpec((H, 3 * H), lambda i: (0, 0))
    bs_b = pl.BlockSpec((1, 3 * H), lambda i: (0, 0))
    bs_tab = pl.BlockSpec((2, B, HH), lambda i: (0, i, 0))
    out_shape = [jax.ShapeDtypeStruct((2, N, HH), jnp.float32),
                 jax.ShapeDtypeStruct((N, H), jnp.float32)]
    return pl.pallas_call(
        _gru_body, grid=grid,
        in_specs=[bs_h, bs_h, bs_f, bs_h, bs_h, bs_f,
                  bs_wih, bs_wih, bs_whh, bs_b, bs_b,
                  bs_wih, bs_wih, bs_whh, bs_b, bs_b],
        out_specs=[bs_tab, bs_f],
        out_shape=out_shape,
    )(agl, agh, xg, asl, ash, xs, *wg, *ws)


def _sage_body(s1l, s1h, c1, s2l, s2h, c2, state,
               Wl1T_lo, Wl1T_hi, Wr1T, bl1,
               Wl2T_lo, Wl2T_hi, Wr2T, bl2,
               W1T, b1, W2T, b2, out):
    cnt1 = jnp.maximum(c1[:, 0:1], 1.0)
    m1l = s1l[:] / cnt1
    m1h = s1h[:] / cnt1
    common1 = jax.nn.relu(
        jnp.dot(m1l, Wl1T_lo[:], preferred_element_type=jnp.float32)
        + jnp.dot(m1h, Wl1T_hi[:], preferred_element_type=jnp.float32)
        + bl1[:]
        + jnp.dot(state[:], Wr1T[:], preferred_element_type=jnp.float32))
    cnt2 = jnp.maximum(c2[:, 0:1], 1.0)
    m2l = s2l[:] / cnt2
    m2h = s2h[:] / cnt2
    common2 = jax.nn.relu(
        jnp.dot(m2l, Wl2T_lo[:], preferred_element_type=jnp.float32)
        + jnp.dot(m2h, Wl2T_hi[:], preferred_element_type=jnp.float32)
        + bl2[:]
        + jnp.dot(common1, Wr2T[:], preferred_element_type=jnp.float32))
    h = jax.nn.relu(jnp.dot(common2, W1T[:], preferred_element_type=jnp.float32)
                    + b1[:])
    out[:] = jnp.dot(h, W2T[:], preferred_element_type=jnp.float32) + b2[:]


def _tc_sage(s1l, s1h, c1, s2l, s2h, c2, state, weights):
    B = 1000
    grid = (N // B,)
    bs_h = pl.BlockSpec((B, HH), lambda i: (i, 0))
    bs_c = pl.BlockSpec((B, HH), lambda i: (i, 0))
    bs_f = pl.BlockSpec((B, H), lambda i: (i, 0))
    bs_whalf = pl.BlockSpec((HH, H), lambda i: (0, 0))
    bs_wfull = pl.BlockSpec((H, H), lambda i: (0, 0))
    bs_b = pl.BlockSpec((1, H), lambda i: (0, 0))
    bs_w1 = pl.BlockSpec((H, 20), lambda i: (0, 0))
    bs_b1 = pl.BlockSpec((1, 20), lambda i: (0, 0))
    bs_w2 = pl.BlockSpec((20, 1), lambda i: (0, 0))
    bs_b2 = pl.BlockSpec((1, 1), lambda i: (0, 0))
    bs_out = pl.BlockSpec((B, 1), lambda i: (i, 0))
    return pl.pallas_call(
        _sage_body, grid=grid,
        in_specs=[bs_h, bs_h, bs_c, bs_h, bs_h, bs_c, bs_f,
                  bs_whalf, bs_whalf, bs_wfull, bs_b,
                  bs_whalf, bs_whalf, bs_wfull, bs_b,
                  bs_w1, bs_b1, bs_w2, bs_b2],
        out_specs=bs_out,
        out_shape=jax.ShapeDtypeStruct((N, 1), jnp.float32),
    )(s1l, s1h, c1, s2l, s2h, c2, state, *weights)


def kernel(x_game, x_state, W_g, Wih_g, Whh_g, bih_g, bhh_g,
           W_s, Wih_s, Whh_s, bih_s, bhh_s,
           Wl1, bl1, Wr1, Wl2, bl2, Wr2,
           W1, b1, W2, b2,
           edge_index_gg, edge_index_ss, edge_index_hist, edge_index_in):
    f32 = jnp.float32
    consts = (jnp.zeros((R, HH), f32), jnp.ones((CHUNK, HH), f32))

    mg, ms = _tc_mm(x_game, x_state, W_g, W_s)

    agg_g, agg_s = _segment_sums(
        mg, ms, edge_index_gg, edge_index_ss, consts)

    wg = (Wih_g.T[:HH], Wih_g.T[HH:], Whh_g.T,
          bih_g.reshape(1, -1), bhh_g.reshape(1, -1))
    ws = (Wih_s.T[:HH], Wih_s.T[HH:], Whh_s.T,
          bih_s.reshape(1, -1), bhh_s.reshape(1, -1))
    game_tab, state_x = _tc_gru(
        agg_g[0, :N], agg_g[1, :N], x_game,
        agg_s[0, :N], agg_s[1, :N], x_state, wg, ws)

    sum1, sum2, cntO = _segment_sums(
        game_tab, game_tab, edge_index_hist, edge_index_in, consts,
        with_counts=True)
    s1l, s1h = sum1[0], sum1[1]
    s2l, s2h = sum2[0], sum2[1]
    c1, c2 = cntO[0], cntO[1]

    weights = (Wl1.T[:HH], Wl1.T[HH:], Wr1.T, bl1.reshape(1, -1),
               Wl2.T[:HH], Wl2.T[HH:], Wr2.T, bl2.reshape(1, -1),
               W1.T, b1.reshape(1, -1), W2.T, b2.reshape(1, -1))
    return _tc_sage(s1l[:N], s1h[:N], c1[:N], s2l[:N], s2h[:N], c2[:N],
                    state_x, weights)
